# Initial kernel scaffold; baseline (speedup 1.0000x reference)
import jax
import jax.numpy as jnp
from jax.experimental import pallas as pl


def kernel(X, edge_index, W1, b1, W2, b2, W3, b3, alphas, gammas, core, f0, f1, f2, f3, tbias):
    raise NotImplementedError("write your pallas kernel here")


# speedup vs baseline: 44.1518x; 44.1518x over previous
import functools

import jax
import jax.numpy as jnp
from jax import lax
from jax.experimental import pallas as pl
from jax.experimental.pallas import tpu as pltpu
from jax.experimental.pallas import tpu_sc as plsc

A_, B_, L_, R_ = 1.0, 1.0, -1.0, 1.0
_LANES = 16
_NCORES = 2
_NSUB = 16


_NW = _NCORES * _NSUB


@functools.lru_cache(maxsize=None)
def _make_sc_agg(NP, EPW, CH, n_chunks):
  mesh = plsc.VectorSubcoreMesh(core_axis_name="c", subcore_axis_name="s",
                                num_cores=_NCORES, num_subcores=_NSUB)

  FL = NP * _LANES
  C4 = 3 * CH

  @functools.partial(
      pl.kernel,
      out_type=jax.ShapeDtypeStruct((_NW * FL,), jnp.float32),
      mesh=mesh,
      scratch_types=[
          pltpu.VMEM((C4,), jnp.int32),
          pltpu.VMEM((C4,), jnp.int32),
          pltpu.VMEM((CH, 128), jnp.float32),
          pltpu.VMEM((CH, 128), jnp.float32),
          pltpu.VMEM((FL,), jnp.float32),
          pltpu.SemaphoreType.DMA,
          pltpu.SemaphoreType.DMA,
          pltpu.SemaphoreType.DMA,
          pltpu.SemaphoreType.DMA,
      ],
  )
  def agg(pk_hbm, w_hbm, zero_hbm, out_hbm,
          ib0, ib1, rb0, rb1, acc_v, is0, is1, gs0, gs1):
    cid = lax.axis_index("c")
    sid = lax.axis_index("s")
    wid = sid * _NCORES + cid
    cbase = wid * jnp.int32(n_chunks)
    pltpu.sync_copy(zero_hbm, acc_v)
    ib = (ib0, ib1)
    rb = (rb0, rb1)
    isem = (is0, is1)
    gsem = (gs0, gs1)

    def idx_start(c, b):
      pltpu.async_copy(pk_hbm.at[pl.ds((cbase + c) * jnp.int32(C4), C4)],
                       ib[b], isem[b])

    def idx_wait(c, b):
      pltpu.make_async_copy(
          pk_hbm.at[pl.ds((cbase + c) * jnp.int32(C4), C4)],
          ib[b], isem[b]).wait()

    def gather_start(b):
      pltpu.async_copy(w_hbm.at[ib[b].at[pl.ds(0, CH)]], rb[b], gsem[b])

    def gather_wait(b):
      pltpu.make_async_copy(w_hbm.at[ib[b].at[pl.ds(0, CH)]],
                            rb[b], gsem[b]).wait()

    idx_start(jnp.int32(0), 0)
    idx_wait(jnp.int32(0), 0)
    gather_start(0)
    idx_start(jnp.int32(1), 1)

    def pair(i2, carry):
      for half in range(2):
        b = half
        nb = 1 - half
        i = i2 * jnp.int32(2) + jnp.int32(half)
        gather_wait(b)

        @pl.when(i + 1 < jnp.int32(n_chunks))
        def _next_gather():
          idx_wait(i + 1, nb)
          gather_start(nb)

        ovecs = []
        svecs = []
        for g in range(CH // _LANES):
          j0 = g * _LANES
          ovecs.append(ib[b][pl.ds(CH + j0, _LANES)])
          svecs.append(ib[b][pl.ds(2 * CH + j0, _LANES)] * jnp.int32(_LANES))

        @pl.when(i + 2 < jnp.int32(n_chunks))
        def _next_idx():
          idx_start(i + 2, b)

        for g in range(CH // _LANES):
          for k in range(_LANES):
            acc_v[pl.ds(svecs[g][k], _LANES)] += (
                rb[b][g * _LANES + k, pl.ds(ovecs[g][k], _LANES)])
      return carry

    lax.fori_loop(jnp.int32(0), jnp.int32(n_chunks // 2), pair, jnp.int32(0))
    pltpu.sync_copy(acc_v, out_hbm.at[pl.ds(wid * jnp.int32(FL), FL)])

  return agg


@functools.lru_cache(maxsize=None)
def _make_sc_deg(NP, EPW, CH, n_chunks):
  mesh = plsc.VectorSubcoreMesh(core_axis_name="c", subcore_axis_name="s",
                                num_cores=_NCORES, num_subcores=_NSUB)

  FL = NP * _LANES

  @functools.partial(
      pl.kernel,
      out_type=jax.ShapeDtypeStruct((_NW * FL,), jnp.float32),
      mesh=mesh,
      scratch_types=[
          pltpu.VMEM((CH,), jnp.int32),
          pltpu.VMEM((CH,), jnp.int32),
          pltpu.VMEM((FL,), jnp.float32),
          pltpu.SemaphoreType.DMA,
          pltpu.SemaphoreType.DMA,
      ],
  )
  def deg(pk_hbm, zero_hbm, out_hbm, ib0, ib1, acc_v, is0, is1):
    cid = lax.axis_index("c")
    sid = lax.axis_index("s")
    wid = sid * _NCORES + cid
    cbase = wid * jnp.int32(n_chunks)
    pltpu.sync_copy(zero_hbm, acc_v)
    ones = jnp.full((_LANES,), 1.0, jnp.float32)
    ib = (ib0, ib1)
    isem = (is0, is1)

    def idx_start(c, b):
      pltpu.async_copy(pk_hbm.at[pl.ds((cbase + c) * jnp.int32(CH), CH)],
                       ib[b], isem[b])

    def idx_wait(c, b):
      pltpu.make_async_copy(
          pk_hbm.at[pl.ds((cbase + c) * jnp.int32(CH), CH)],
          ib[b], isem[b]).wait()

    idx_start(jnp.int32(0), 0)
    idx_start(jnp.int32(1), 1)

    def pair(i2, carry):
      for half in range(2):
        b = half
        i = i2 * jnp.int32(2) + jnp.int32(half)
        idx_wait(i, b)

        def group(g, c2):
          j0 = g * jnp.int32(_LANES)
          dvec = ib[b][pl.ds(j0, _LANES)]
          for k in range(_LANES):
            acc_v[pl.ds(dvec[k], _LANES)] += ones
          return c2

        lax.fori_loop(jnp.int32(0), jnp.int32(CH // _LANES), group,
                      jnp.int32(0))

        @pl.when(i + 2 < jnp.int32(n_chunks))
        def _next_idx():
          idx_start(i + 2, b)
      return carry

    lax.fori_loop(jnp.int32(0), jnp.int32(n_chunks // 2), pair, jnp.int32(0))
    pltpu.sync_copy(acc_v, out_hbm.at[pl.ds(wid * jnp.int32(FL), FL)])

  return deg


def _psum(flat_ref, nw, fl):
  acc = flat_ref[pl.ds(0, fl)]
  for i in range(1, nw):
    acc = acc + flat_ref[pl.ds(i * fl, fl)]
  return acc


def _prep_body(nw, fl, g0_ref, degp_ref, f0_ref, norm_ref, w0_ref, gacc_ref):
  deg = _psum(degp_ref, nw, fl)
  norm = jnp.where(deg > 0.0, lax.rsqrt(jnp.maximum(deg, 1.0)), 0.0)
  norm_ref[...] = norm
  w0_ref[...] = norm * f0_ref[...]
  gacc_ref[...] = g0_ref[0] * f0_ref[...]


def _step_body(nw, fl, scal_ref, aggp_ref, norm_ref, ysp_ref, ysp2_ref,
               gin_ref, ys_ref, gout_ref, w_ref):
  agg = _psum(aggp_ref, nw, fl)
  norm = norm_ref[...]
  ys = (scal_ref[0] * norm * agg + scal_ref[1] * ysp_ref[...]
        + scal_ref[2] * ysp2_ref[...])
  ys_ref[...] = ys
  gout_ref[...] = gin_ref[...] + scal_ref[3] * ys
  w_ref[...] = norm * ys


def _mm_body(x_ref, g_ref, h_ref, u_ref):
  i = pl.program_id(0)

  @pl.when(i == 0)
  def _init():
    h_ref[...] = jnp.zeros_like(h_ref)
    u_ref[...] = jnp.zeros_like(u_ref)

  xb = x_ref[...]
  gb = g_ref[...]
  h_ref[...] += lax.dot_general(xb, gb, (((0,), (0,)), ((), ())),
                                preferred_element_type=jnp.float32)
  colsum = jnp.sum(gb, axis=0, keepdims=True)
  u_ref[...] += jnp.broadcast_to(colsum, u_ref.shape)


def _epi_body(f2_ref, tb_ref, h_ref, u_ref, w1_ref, b1_ref, b2_ref, b3_ref,
              f1_ref, coret_ref, f3_ref, out_ref):
  H8 = h_ref[:, 0:8]
  u = u_ref[0:1, 0:8]
  dn = (((0,), (0,)), ((), ()))
  S0 = lax.dot_general(w1_ref[...], H8, dn,
                       preferred_element_type=jnp.float32)
  S0 = S0 + b1_ref[...] * u
  Q0 = lax.dot_general(S0, f1_ref[...], dn,
                       preferred_element_type=jnp.float32)
  fb2 = lax.dot_general(b2_ref[...], f1_ref[...], dn,
                        preferred_element_type=jnp.float32)
  fb3 = lax.dot_general(b3_ref[...], f1_ref[...], dn,
                        preferred_element_type=jnp.float32)
  Q1 = lax.dot_general(u, fb2, dn, preferred_element_type=jnp.float32)
  Q2 = lax.dot_general(u, fb3, dn, preferred_element_type=jnp.float32)
  v = jnp.zeros((1, 8), jnp.float32)
  for b in range(3):
    for g in range(3):
      qb = (f2_ref[0, g] * Q0[:, b:b + 1] + f2_ref[1, g] * Q1[:, b:b + 1]
            + f2_ref[2, g] * Q2[:, b:b + 1])
      cb = coret_ref[(b * 3 + g) * 8:(b * 3 + g + 1) * 8, :]
      v = v + lax.dot_general(qb, cb, dn,
                              preferred_element_type=jnp.float32)
  o = lax.dot_general(v, f3_ref[...], (((1,), (1,)), ((), ())),
                      preferred_element_type=jnp.float32) + tb_ref[0, 0]
  o = o - jnp.max(o, axis=-1, keepdims=True)
  out_ref[...] = o - jnp.log(jnp.sum(jnp.exp(o), axis=-1, keepdims=True))


def kernel(X, edge_index, W1, b1, W2, b2, W3, b3, alphas, gammas, core,
           f0, f1, f2, f3, tbias):
  N, D = X.shape
  E = edge_index.shape[1]
  K = alphas.shape[0] - 1
  NP = ((N + 8 + 7) // 8) * 8
  R2 = NP * _LANES // 128
  CH = 32
  CHD = 256
  NW = _NW
  epw_raw = -(-E // NW)

  def _grid(ch):
    nc = -(-epw_raw // ch)
    nc += nc % 2
    return nc, nc * ch * NW

  n_chunks, EPAD = _grid(CH)
  n_chunks_d, EPAD_D = _grid(CHD)

  src = edge_index[0].astype(jnp.int32)
  dst = edge_index[1].astype(jnp.int32)

  def _padded(a, epad):
    return jnp.concatenate([a, jnp.full((epad - E,), N, jnp.int32)])

  srcp = _padded(src, EPAD)
  dstp = _padded(dst, EPAD)
  dstd = _padded(dst, EPAD_D)
  TOTC = EPAD // CH
  pk = jnp.stack([(dstp // 8).reshape(TOTC, CH),
                  (dstp % 8 * _LANES).reshape(TOTC, CH),
                  srcp.reshape(TOTC, CH)], axis=1).reshape(-1)
  pkd = dstd * _LANES

  FL = NP * _LANES
  zero_t = jnp.zeros((FL,), jnp.float32)
  f0p = jnp.pad(f0.astype(jnp.float32), ((0, NP - N), (0, _LANES - 8)))
  f0f = f0p.reshape(FL)

  agg_fn = _make_sc_agg(NP, 0, CH, n_chunks)
  deg_fn = _make_sc_deg(NP, 0, CHD, n_chunks_d)

  degp = deg_fn(pkd, zero_t)

  shf = jax.ShapeDtypeStruct((FL,), jnp.float32)
  norm16, w, gacc = pl.pallas_call(
      functools.partial(_prep_body, NW, FL),
      out_shape=(shf, shf, shf),
      in_specs=[pl.BlockSpec(memory_space=pltpu.SMEM),
                pl.BlockSpec(memory_space=pltpu.VMEM),
                pl.BlockSpec(memory_space=pltpu.VMEM)],
  )(gammas[0:1].astype(jnp.float32), degp, f0f)

  al = jnp.tanh(alphas.astype(jnp.float32))
  gam = gammas.astype(jnp.float32)
  c1 = (A_ - B_) / 2.0 - (A_ + B_ + 2.0) / 2.0 * (L_ + R_) / (R_ - L_)
  c2 = (A_ + B_ + 2.0) / (R_ - L_)
  scals = [jnp.stack([al[0] * c2, al[0] * c1,
                      jnp.zeros((), jnp.float32), gam[1]]).astype(jnp.float32)]
  for Lk in range(2, K + 1):
    coef_l = 2.0 * Lk * (Lk + A_ + B_) * (2.0 * Lk - 2.0 + A_ + B_)
    coef_lm1_1 = ((2.0 * Lk + A_ + B_ - 1.0) * (2.0 * Lk + A_ + B_)
                  * (2.0 * Lk + A_ + B_ - 2.0))
    coef_lm1_2 = (2.0 * Lk + A_ + B_ - 1.0) * (A_ * A_ - B_ * B_)
    coef_lm2 = 2.0 * (Lk - 1.0 + A_) * (Lk - 1.0 + B_) * (2.0 * Lk + A_ + B_)
    tmp1 = al[Lk - 1] * (coef_lm1_1 / coef_l)
    tmp2 = al[Lk - 1] * (coef_lm1_2 / coef_l)
    tmp3 = al[Lk - 1] * al[Lk - 2] * (coef_lm2 / coef_l)
    tmp1_2 = tmp1 * (2.0 / (R_ - L_))
    tmp2_2 = tmp1 * ((R_ + L_) / (R_ - L_)) + tmp2
    scals.append(jnp.stack([tmp1_2, -tmp2_2, -tmp3,
                            gam[Lk]]).astype(jnp.float32))

  ys_prev2 = f0f
  ys_prev = f0f
  for k in range(1, K + 1):
    aggp = agg_fn(pk, w.reshape(R2, 128), zero_t)
    ys_new, gacc, w = pl.pallas_call(
        functools.partial(_step_body, NW, FL),
        out_shape=(shf, shf, shf),
        in_specs=[pl.BlockSpec(memory_space=pltpu.SMEM)] +
                 [pl.BlockSpec(memory_space=pltpu.VMEM)] * 5,
    )(scals[k - 1], aggp, norm16, ys_prev, ys_prev2, gacc)
    ys_prev2, ys_prev = ys_prev, ys_new

  NB = 512
  NG = -(-N // NB)
  NPAD = NG * NB
  Xp = jnp.pad(X, ((0, NPAD - N), (0, 0)))
  G2 = gacc.reshape(NP, _LANES)[:N]
  Gp = jnp.pad(G2, ((0, NPAD - N), (0, 0)))
  H, U = pl.pallas_call(
      _mm_body,
      grid=(NG,),
      out_shape=(jax.ShapeDtypeStruct((D, _LANES), jnp.float32),
                 jax.ShapeDtypeStruct((8, _LANES), jnp.float32)),
      in_specs=[pl.BlockSpec((NB, D), lambda i: (i, jnp.int32(0))),
                pl.BlockSpec((NB, _LANES), lambda i: (i, jnp.int32(0)))],
      out_specs=(pl.BlockSpec((D, _LANES),
                              lambda i: (jnp.int32(0), jnp.int32(0))),
                 pl.BlockSpec((8, _LANES),
                              lambda i: (jnp.int32(0), jnp.int32(0)))),
  )(Xp, Gp)

  core_t = jnp.transpose(core, (1, 2, 0, 3)).reshape(72, 8)
  out = pl.pallas_call(
      _epi_body,
      out_shape=jax.ShapeDtypeStruct((1, 8), jnp.float32),
      in_specs=[pl.BlockSpec(memory_space=pltpu.SMEM),
                pl.BlockSpec(memory_space=pltpu.SMEM)] +
               [pl.BlockSpec(memory_space=pltpu.VMEM)] * 9,
  )(f2, tbias.reshape(1, 1), H, U, W1, b1[:, None], b2[:, None],
    b3[:, None], f1, core_t, f3)
  return out
